# baseline (speedup 1.0000x reference)
import jax
import jax.numpy as jnp
from jax.experimental import pallas as pl


def _readout_kernel(x_ref, w_ref, b_ref, o_ref):
    o_ref[:, :] = jnp.sum(x_ref[:, :] * w_ref[0, :], axis=1, keepdims=True) + b_ref[0, 0]


def kernel(pos, x, W, b):
    B, N, F = x.shape
    R = B * N
    x2 = x.reshape(R, F)
    BLK = 2048
    grid = (R // BLK,)
    out = pl.pallas_call(
        _readout_kernel,
        grid=grid,
        in_specs=[
            pl.BlockSpec((BLK, F), lambda i: (i, 0)),
            pl.BlockSpec((1, F), lambda i: (0, 0)),
            pl.BlockSpec((1, 1), lambda i: (0, 0)),
        ],
        out_specs=pl.BlockSpec((BLK, 1), lambda i: (i, 0)),
        out_shape=jax.ShapeDtypeStruct((R, 1), jnp.float32),
    )(x2, W, b.reshape(1, 1))
    return out.reshape(B, N, 1)
